# baseline (speedup 1.0000x reference)
import jax
import jax.numpy as jnp
from jax import lax
from jax.experimental import pallas as pl
from jax.experimental.pallas import tpu as pltpu
from jax.experimental.pallas import tpu_sc as plsc

N = 10000
E = 320000
D = 128

NC = 2
NS = 16
NW = NC * NS
CH = 128
NB = 2
SCH = 16
NSUP = 160
M0 = 5
M1 = 5
MD = NSUP // NW
E_PAD = NSUP * SCH * CH
N_ACC = 10112
R_STRIPE = N_ACC // NS

_MESH = plsc.VectorSubcoreMesh(
    core_axis_name="c", subcore_axis_name="s", num_cores=NC, num_subcores=NS
)


def _deg_body(dst_hbm, ones_hbm, z_hbm, out_hbm, dst_v, ones_v, acc_sh, sem):
    c = lax.axis_index("c")
    s = lax.axis_index("s")
    r0 = s * R_STRIPE
    pltpu.sync_copy(z_hbm.at[pl.ds(r0, R_STRIPE)], acc_sh.at[pl.ds(r0, R_STRIPE)])
    pltpu.sync_copy(ones_hbm, ones_v)
    w = c * NS + s
    for m in range(MD):
        pltpu.sync_copy(dst_hbm.at[w * MD + m], dst_v.at[pl.ds(m * SCH, SCH)])
    plsc.subcore_barrier()

    @pl.loop(0, MD * SCH, step=8)
    def _(j0):
        for b in range(8):
            pltpu.async_copy(ones_v, acc_sh.at[dst_v.at[j0 + b]], sem, add=True)
        for b in range(8):
            pltpu.make_async_copy(ones_v, acc_sh.at[dst_v.at[j0 + b]], sem).wait()

    plsc.subcore_barrier()
    pltpu.sync_copy(acc_sh.at[pl.ds(r0, R_STRIPE)], out_hbm.at[c, pl.ds(r0, R_STRIPE)])


def _sc_degree(dst3, ones128, z128):
    k = pl.kernel(
        _deg_body,
        out_type=jax.ShapeDtypeStruct((NC, N_ACC, D), jnp.float32),
        mesh=_MESH,
        scratch_types=[
            pltpu.VMEM((MD * SCH, CH), jnp.int32),
            pltpu.VMEM((CH, D), jnp.float32),
            pltpu.VMEM_SHARED((N_ACC, D), jnp.float32),
            pltpu.SemaphoreType.DMA,
        ],
    )
    return k(dst3, ones128, z128)


def _gather_body(g_hbm, src_hbm, dst_hbm, z_hbm, out_hbm,
                 src_v, dst_v, rows0, acc_sh, sem):
    c = lax.axis_index("c")
    s = lax.axis_index("s")
    r0 = s * R_STRIPE
    pltpu.sync_copy(z_hbm.at[pl.ds(r0, R_STRIPE)], acc_sh.at[pl.ds(r0, R_STRIPE)])
    w = c * NS + s
    for m in range(MD):
        pltpu.sync_copy(src_hbm.at[w * MD + m], src_v.at[pl.ds(m * SCH, SCH)])
        pltpu.sync_copy(dst_hbm.at[w * MD + m], dst_v.at[pl.ds(m * SCH, SCH)])
    plsc.subcore_barrier()

    @pl.loop(0, MD * SCH)
    def _(j):
        pltpu.sync_copy(g_hbm.at[src_v.at[j]], rows0)
        pltpu.sync_copy(rows0, acc_sh.at[dst_v.at[j]], add=True)

    plsc.subcore_barrier()
    pltpu.sync_copy(acc_sh.at[pl.ds(r0, R_STRIPE)], out_hbm.at[c, pl.ds(r0, R_STRIPE)])


def _sc_segment_sum(g, src3, dst3, z128):
    k = pl.kernel(
        _gather_body,
        out_type=jax.ShapeDtypeStruct((NC, N_ACC, D), jnp.float32),
        mesh=_MESH,
        scratch_types=[
            pltpu.VMEM((MD * SCH, CH), jnp.int32),
            pltpu.VMEM((MD * SCH, CH), jnp.int32),
            pltpu.VMEM((CH, D), jnp.float32),
            pltpu.VMEM_SHARED((N_ACC, D), jnp.float32),
            pltpu.SemaphoreType.DMA,
        ],
    )
    return k(g, src3, dst3, z128)


def _dot(a, b):
    return jnp.dot(a, b, preferred_element_type=jnp.float32,
                   precision=lax.Precision.HIGHEST)


BN = 2000


def _dinv_from(degp_ref):
    deg = 1.0 + degp_ref[0, :, 0:1] + degp_ref[1, :, 0:1]
    return lax.rsqrt(deg)


def _row_specs():
    return dict(
        grid=(N // BN,),
        out_specs=pl.BlockSpec((BN, D), lambda i: (i, 0)),
    )


def _mm_body(x_ref, w_ref, o_ref):
    o_ref[...] = _dot(x_ref[...], w_ref[...])


def _tc_matmul(x, w):
    return pl.pallas_call(
        _mm_body,
        out_shape=jax.ShapeDtypeStruct((x.shape[0], w.shape[1]), jnp.float32),
    )(x, w)


def _scale_body(h_ref, degp_ref, o_ref):
    o_ref[...] = h_ref[...] * _dinv_from(degp_ref)


def _tc_scale(h, degp):
    return pl.pallas_call(
        _scale_body,
        out_shape=jax.ShapeDtypeStruct((N, D), jnp.float32),
        in_specs=[
            pl.BlockSpec((BN, D), lambda i: (i, 0)),
            pl.BlockSpec((NC, BN, D), lambda i: (0, i, 0)),
        ],
        **_row_specs(),
    )(h, degp)


def _mid_body(p_ref, g_ref, degp_ref, b_ref, w_ref, o_ref):
    dinv = _dinv_from(degp_ref)
    acc = p_ref[0] + p_ref[1] + g_ref[...]
    out1 = jnp.maximum(dinv * acc + b_ref[...], 0.0)
    o_ref[...] = _dot(out1, w_ref[...]) * dinv


def _tc_mid(p, g, degp, b1, w2):
    return pl.pallas_call(
        _mid_body,
        out_shape=jax.ShapeDtypeStruct((N, D), jnp.float32),
        in_specs=[
            pl.BlockSpec((NC, BN, D), lambda i: (0, i, 0)),
            pl.BlockSpec((BN, D), lambda i: (i, 0)),
            pl.BlockSpec((NC, BN, D), lambda i: (0, i, 0)),
            pl.BlockSpec((1, D), lambda i: (0, 0)),
            pl.BlockSpec((D, D), lambda i: (0, 0)),
        ],
        **_row_specs(),
    )(p, g, degp, b1.reshape(1, D), w2)


def _final_body(p_ref, g_ref, degp_ref, b_ref, o_ref):
    dinv = _dinv_from(degp_ref)
    acc = p_ref[0] + p_ref[1] + g_ref[...]
    o_ref[...] = dinv * acc + b_ref[...]


def _tc_final(p, g, degp, b2):
    return pl.pallas_call(
        _final_body,
        out_shape=jax.ShapeDtypeStruct((N, D), jnp.float32),
        in_specs=[
            pl.BlockSpec((NC, BN, D), lambda i: (0, i, 0)),
            pl.BlockSpec((BN, D), lambda i: (i, 0)),
            pl.BlockSpec((NC, BN, D), lambda i: (0, i, 0)),
            pl.BlockSpec((1, D), lambda i: (0, 0)),
        ],
        **_row_specs(),
    )(p, g, degp, b2.reshape(1, D))


@jax.jit
def kernel(x, edge_index, W1, b1, W2, b2):
    pad = E_PAD - E
    src3 = jnp.concatenate(
        [edge_index[0], jnp.zeros((pad,), jnp.int32)]).reshape(NSUP, SCH, CH)
    junk = N + jnp.arange(pad, dtype=jnp.int32) % (N_ACC - N)
    dst3 = jnp.concatenate([edge_index[1], junk]).reshape(NSUP, SCH, CH)
    ones128 = jnp.ones((CH, D), jnp.float32)
    z128 = jnp.zeros((N_ACC, D), jnp.float32)

    degp = _sc_degree(dst3, ones128, z128)
    h1 = _tc_matmul(x, W1)

    g1 = _tc_scale(h1, degp)
    p1 = _sc_segment_sum(g1, src3, dst3, z128)
    g2 = _tc_mid(p1, g1, degp, b1, W2)
    p2 = _sc_segment_sum(g2, src3, dst3, z128)
    return _tc_final(p2, g2, degp, b2)


# speedup vs baseline: 1.4819x; 1.4819x over previous
import jax
import jax.numpy as jnp
from jax import lax
from jax.experimental import pallas as pl
from jax.experimental.pallas import tpu as pltpu
from jax.experimental.pallas import tpu_sc as plsc

N = 10000
E = 320000
D = 128

NC = 2
NS = 16
NW = NC * NS
CH = 128
CPW = -(-E // (NW * CH))
E_PAD = NW * CPW * CH
JUNK = 10008
N_ACC = 10112
R_STRIPE = N_ACC // NS

_MESH = plsc.VectorSubcoreMesh(
    core_axis_name="c", subcore_axis_name="s", num_cores=NC, num_subcores=NS
)


def _deg_body(dst_hbm, ones_hbm, z_hbm, out_hbm, dst_v, ones_v, acc_sh, sem):
    c = lax.axis_index("c")
    s = lax.axis_index("s")
    r0 = s * R_STRIPE
    pltpu.sync_copy(z_hbm.at[pl.ds(r0, R_STRIPE)], acc_sh.at[pl.ds(r0, R_STRIPE)])
    pltpu.sync_copy(ones_hbm, ones_v)
    w = c * NS + s
    pltpu.sync_copy(dst_hbm.at[w], dst_v)
    plsc.subcore_barrier()

    @pl.loop(0, CPW)
    def _(j):
        pltpu.sync_copy(ones_v, acc_sh.at[dst_v.at[j]], add=True)

    plsc.subcore_barrier()
    pltpu.sync_copy(acc_sh.at[pl.ds(r0, R_STRIPE)], out_hbm.at[c, pl.ds(r0, R_STRIPE)])


def _sc_degree(dst_pad, ones128, z128):
    k = pl.kernel(
        _deg_body,
        out_type=jax.ShapeDtypeStruct((NC, N_ACC, D), jnp.float32),
        mesh=_MESH,
        scratch_types=[
            pltpu.VMEM((CPW, CH), jnp.int32),
            pltpu.VMEM((CH, D), jnp.float32),
            pltpu.VMEM_SHARED((N_ACC, D), jnp.float32),
            pltpu.SemaphoreType.DMA,
        ],
    )
    return k(dst_pad, ones128, z128)


def _gather_body(g_hbm, src_hbm, dst_hbm, z_hbm, out_hbm,
                 src_v, dst_v, rows_v, acc_sh, sem):
    c = lax.axis_index("c")
    s = lax.axis_index("s")
    r0 = s * R_STRIPE
    pltpu.sync_copy(z_hbm.at[pl.ds(r0, R_STRIPE)], acc_sh.at[pl.ds(r0, R_STRIPE)])
    w = c * NS + s
    pltpu.sync_copy(src_hbm.at[w], src_v)
    pltpu.sync_copy(dst_hbm.at[w], dst_v)
    plsc.subcore_barrier()

    @pl.loop(0, CPW)
    def _(j):
        pltpu.sync_copy(g_hbm.at[src_v.at[j]], rows_v)
        pltpu.sync_copy(rows_v, acc_sh.at[dst_v.at[j]], add=True)

    plsc.subcore_barrier()
    pltpu.sync_copy(acc_sh.at[pl.ds(r0, R_STRIPE)], out_hbm.at[c, pl.ds(r0, R_STRIPE)])


def _sc_segment_sum(g, src_pad, dst_pad, z128):
    k = pl.kernel(
        _gather_body,
        out_type=jax.ShapeDtypeStruct((NC, N_ACC, D), jnp.float32),
        mesh=_MESH,
        scratch_types=[
            pltpu.VMEM((CPW, CH), jnp.int32),
            pltpu.VMEM((CPW, CH), jnp.int32),
            pltpu.VMEM((CH, D), jnp.float32),
            pltpu.VMEM_SHARED((N_ACC, D), jnp.float32),
            pltpu.SemaphoreType.DMA,
        ],
    )
    return k(g, src_pad, dst_pad, z128)


def _dot(a, b):
    return jnp.dot(a, b, preferred_element_type=jnp.float32,
                   precision=lax.Precision.HIGHEST)


def _dinv_from(degp_ref):
    deg = 1.0 + degp_ref[0, :, 0:1] + degp_ref[1, :, 0:1]
    return lax.rsqrt(deg)[:N]


def _mm_body(x_ref, w_ref, o_ref):
    o_ref[...] = _dot(x_ref[...], w_ref[...])


def _tc_matmul(x, w):
    return pl.pallas_call(
        _mm_body,
        out_shape=jax.ShapeDtypeStruct((x.shape[0], w.shape[1]), jnp.float32),
    )(x, w)


def _scale_body(h_ref, degp_ref, o_ref):
    o_ref[...] = h_ref[...] * _dinv_from(degp_ref)


def _tc_scale(h, degp):
    return pl.pallas_call(
        _scale_body,
        out_shape=jax.ShapeDtypeStruct((N, D), jnp.float32),
    )(h, degp)


def _mid_body(p_ref, g_ref, degp_ref, b_ref, w_ref, o_ref):
    dinv = _dinv_from(degp_ref)
    acc = p_ref[0, :N, :] + p_ref[1, :N, :] + g_ref[...]
    out1 = jnp.maximum(dinv * acc + b_ref[...], 0.0)
    o_ref[...] = _dot(out1, w_ref[...]) * dinv


def _tc_mid(p, g, degp, b1, w2):
    return pl.pallas_call(
        _mid_body,
        out_shape=jax.ShapeDtypeStruct((N, D), jnp.float32),
    )(p, g, degp, b1.reshape(1, D), w2)


def _final_body(p_ref, g_ref, degp_ref, b_ref, o_ref):
    dinv = _dinv_from(degp_ref)
    acc = p_ref[0, :N, :] + p_ref[1, :N, :] + g_ref[...]
    o_ref[...] = dinv * acc + b_ref[...]


def _tc_final(p, g, degp, b2):
    return pl.pallas_call(
        _final_body,
        out_shape=jax.ShapeDtypeStruct((N, D), jnp.float32),
    )(p, g, degp, b2.reshape(1, D))


@jax.jit
def kernel(x, edge_index, W1, b1, W2, b2):
    pad = E_PAD - E
    src_pad = jnp.concatenate(
        [edge_index[0], jnp.zeros((pad,), jnp.int32)]).reshape(NW, CPW, CH)
    dst_pad = jnp.concatenate(
        [edge_index[1], jnp.full((pad,), JUNK, jnp.int32)]).reshape(NW, CPW, CH)
    ones128 = jnp.ones((CH, D), jnp.float32)
    z128 = jnp.zeros((N_ACC, D), jnp.float32)

    degp = _sc_degree(dst_pad, ones128, z128)
    h1 = _tc_matmul(x, W1)

    g1 = _tc_scale(h1, degp)
    p1 = _sc_segment_sum(g1, src_pad, dst_pad, z128)
    g2 = _tc_mid(p1, g1, degp, b1, W2)
    p2 = _sc_segment_sum(g2, src_pad, dst_pad, z128)
    return _tc_final(p2, g2, degp, b2)
